# baseline (speedup 1.0000x reference)
import functools

import jax
import jax.numpy as jnp
from jax import lax
from jax.experimental import pallas as pl
from jax.experimental.pallas import tpu as pltpu
from jax.experimental.pallas import tpu_sc as plsc

N = 100000
E = 1600000
D_IN = 128
HID = 16
D_OUT = 40

NP_ = 102400
NC = 2
NS = 16
NW = NC * NS
ET = E // NW
EC = E // NC
CHUNK_D = 5000
CHUNK_N = 2000
CHUNK_A = 400
NCH_D = ET // CHUNK_D
NCH_N = ET // CHUNK_N
NCH_A = ET // CHUNK_A
ROWS_PT = NP_ // NS

_mesh = plsc.VectorSubcoreMesh(core_axis_name="c", subcore_axis_name="s")


def _f32(*shape):
    return jax.ShapeDtypeStruct(shape, jnp.float32)


@functools.partial(
    pl.kernel,
    out_type=(_f32(NP_), _f32(NP_)),
    mesh=_mesh,
    compiler_params=pltpu.CompilerParams(needs_layout_passes=False, use_tc_tiling_on_sc=False),
    scratch_types=[
        pltpu.VMEM((CHUNK_D,), jnp.int32),
        pltpu.VMEM((CHUNK_D,), jnp.float32),
        pltpu.VMEM_SHARED((NP_,), jnp.float32),
    ],
)
def _deg_kernel(col_hbm, ew_hbm, zeros_hbm, deg0_hbm, deg1_hbm,
                col_v, ew_v, acc):
    c = lax.axis_index("c")
    s = lax.axis_index("s")
    pltpu.sync_copy(zeros_hbm.at[pl.ds(0, ROWS_PT)],
                    acc.at[pl.ds(s * ROWS_PT, ROWS_PT)])
    plsc.subcore_barrier()

    base_t = c * EC + s * ET

    def body(i, carry):
        base = base_t + i * CHUNK_D
        pltpu.sync_copy(col_hbm.at[pl.ds(base, CHUNK_D)], col_v)
        pltpu.sync_copy(ew_hbm.at[pl.ds(base, CHUNK_D)], ew_v)
        pltpu.sync_copy(ew_v, acc.at[col_v], add=True)
        return carry

    lax.fori_loop(0, NCH_D, body, 0)
    plsc.subcore_barrier()

    @pl.when(c == 0)
    def _():
        pltpu.sync_copy(acc.at[pl.ds(s * ROWS_PT, ROWS_PT)],
                        deg0_hbm.at[pl.ds(s * ROWS_PT, ROWS_PT)])

    @pl.when(c == 1)
    def _():
        pltpu.sync_copy(acc.at[pl.ds(s * ROWS_PT, ROWS_PT)],
                        deg1_hbm.at[pl.ds(s * ROWS_PT, ROWS_PT)])


@functools.partial(
    pl.kernel,
    out_type=_f32(E),
    mesh=_mesh,
    compiler_params=pltpu.CompilerParams(needs_layout_passes=False, use_tc_tiling_on_sc=False),
    scratch_types=[
        pltpu.VMEM((NP_,), jnp.float32),
        pltpu.VMEM((CHUNK_N,), jnp.int32),
        pltpu.VMEM((CHUNK_N,), jnp.int32),
        pltpu.VMEM((CHUNK_N,), jnp.float32),
        pltpu.VMEM((CHUNK_N,), jnp.float32),
    ],
)
def _norm_kernel(row_hbm, col_hbm, ew_hbm, dinv_hbm, norm_hbm,
                 dinv_v, row_v, col_v, ew_v, norm_v):
    c = lax.axis_index("c")
    s = lax.axis_index("s")
    pltpu.sync_copy(dinv_hbm, dinv_v)
    base_t = c * EC + s * ET

    def body(i, carry):
        base = base_t + i * CHUNK_N
        pltpu.sync_copy(row_hbm.at[pl.ds(base, CHUNK_N)], row_v)
        pltpu.sync_copy(col_hbm.at[pl.ds(base, CHUNK_N)], col_v)
        pltpu.sync_copy(ew_hbm.at[pl.ds(base, CHUNK_N)], ew_v)

        def inner(g, carry2):
            off = g * 16
            r16 = row_v[pl.ds(off, 16)]
            c16 = col_v[pl.ds(off, 16)]
            e16 = ew_v[pl.ds(off, 16)]
            dr = plsc.load_gather(dinv_v, [r16])
            dc = plsc.load_gather(dinv_v, [c16])
            norm_v[pl.ds(off, 16)] = dr * e16 * dc
            return carry2

        lax.fori_loop(0, CHUNK_N // 16, inner, 0, unroll=4)
        pltpu.sync_copy(norm_v, norm_hbm.at[pl.ds(base, CHUNK_N)])
        return carry

    lax.fori_loop(0, NCH_N, body, 0)


@functools.partial(
    pl.kernel,
    out_type=(_f32(NP_, HID), _f32(NP_, HID)),
    mesh=_mesh,
    compiler_params=pltpu.CompilerParams(needs_layout_passes=False, use_tc_tiling_on_sc=False),
    scratch_types=[
        pltpu.VMEM((CHUNK_A,), jnp.int32),
        pltpu.VMEM((CHUNK_A,), jnp.int32),
        pltpu.VMEM((CHUNK_A,), jnp.float32),
        pltpu.VMEM((CHUNK_A, HID), jnp.float32),
        pltpu.VMEM((CHUNK_A,), jnp.int32),
        pltpu.VMEM((CHUNK_A,), jnp.int32),
        pltpu.VMEM((CHUNK_A,), jnp.float32),
        pltpu.VMEM((CHUNK_A, HID), jnp.float32),
        pltpu.VMEM_SHARED((NP_, HID), jnp.float32),
        pltpu.SemaphoreType.DMA,
        pltpu.SemaphoreType.DMA,
        pltpu.SemaphoreType.DMA,
        pltpu.SemaphoreType.DMA,
    ],
)
def _agg_kernel(row_hbm, col_hbm, norm_hbm, tab_hbm, zeros_hbm,
                out0_hbm, out1_hbm,
                row_v0, col_v0, norm_v0, rows_v0,
                row_v1, col_v1, norm_v1, rows_v1,
                acc, semg0, semg1, sems0, sems1):
    c = lax.axis_index("c")
    s = lax.axis_index("s")
    pltpu.sync_copy(zeros_hbm.at[pl.ds(0, ROWS_PT)],
                    acc.at[pl.ds(s * ROWS_PT, ROWS_PT)])
    plsc.subcore_barrier()

    base_t = c * EC + s * ET
    slots = ((row_v0, col_v0, norm_v0, rows_v0, semg0, sems0),
             (row_v1, col_v1, norm_v1, rows_v1, semg1, sems1))

    def load_and_gather(i, slot):
        row_v, col_v, norm_v, rows_v, semg, _ = slot
        base = base_t + i * CHUNK_A
        pltpu.sync_copy(row_hbm.at[pl.ds(base, CHUNK_A)], row_v)
        pltpu.sync_copy(col_hbm.at[pl.ds(base, CHUNK_A)], col_v)
        pltpu.sync_copy(norm_hbm.at[pl.ds(base, CHUNK_A)], norm_v)
        pltpu.async_copy(tab_hbm.at[row_v], rows_v, semg)

    load_and_gather(0, slots[0])
    load_and_gather(1, slots[1])

    def body(i, carry):
        def step(slot):
            row_v, col_v, norm_v, rows_v, semg, sems = slot
            pltpu.make_async_copy(tab_hbm.at[row_v], rows_v, semg).wait()

            def scale(j, carry2):
                nb = plsc.load_gather(norm_v, [jnp.full((16,), j, jnp.int32)])
                rows_v[j] = rows_v[j] * nb
                return carry2

            lax.fori_loop(0, CHUNK_A, scale, 0, unroll=8)
            pltpu.async_copy(rows_v, acc.at[col_v], sems, add=True)

            @pl.when(i + 2 < NCH_A)
            def _():
                pltpu.make_async_copy(rows_v, acc.at[col_v], sems).wait()
                load_and_gather(i + 2, slot)

        @pl.when(i % 2 == 0)
        def _():
            step(slots[0])

        @pl.when(i % 2 == 1)
        def _():
            step(slots[1])

        return carry

    lax.fori_loop(0, NCH_A, body, 0)
    pltpu.make_async_copy(rows_v0, acc.at[col_v0], sems0).wait()
    pltpu.make_async_copy(rows_v1, acc.at[col_v1], sems1).wait()
    plsc.subcore_barrier()

    @pl.when(c == 0)
    def _():
        pltpu.sync_copy(acc.at[pl.ds(s * ROWS_PT, ROWS_PT)],
                        out0_hbm.at[pl.ds(s * ROWS_PT, ROWS_PT)])

    @pl.when(c == 1)
    def _():
        pltpu.sync_copy(acc.at[pl.ds(s * ROWS_PT, ROWS_PT)],
                        out1_hbm.at[pl.ds(s * ROWS_PT, ROWS_PT)])


def _tc_dinv(deg0, deg1):
    def body(d0_ref, d1_ref, o_ref):
        deg = d0_ref[...] + d1_ref[...] + 1.0
        o_ref[...] = lax.rsqrt(deg)

    return pl.pallas_call(
        body,
        out_shape=jax.ShapeDtypeStruct((NP_ // 128, 128), jnp.float32),
    )(deg0.reshape(NP_ // 128, 128), deg1.reshape(NP_ // 128, 128)).reshape(NP_)


_RB = 5000
_NRB = N // _RB


def _tc_mm1(x, W1):
    def body(x_ref, w_ref, o_ref):
        o_ref[...] = jnp.dot(x_ref[...], w_ref[...],
                             preferred_element_type=jnp.float32)

    return pl.pallas_call(
        body,
        grid=(_NRB,),
        in_specs=[
            pl.BlockSpec((_RB, D_IN), lambda i: (i, 0)),
            pl.BlockSpec((D_IN, HID), lambda i: (0, 0)),
        ],
        out_specs=pl.BlockSpec((_RB, HID), lambda i: (i, 0)),
        out_shape=jax.ShapeDtypeStruct((N, HID), jnp.float32),
    )(x, W1)


def _tc_relu_combine(a0, a1, hlin, dinv, b1):
    def body(a0_ref, a1_ref, h_ref, d_ref, b_ref, o_ref):
        d2 = d_ref[...] * d_ref[...]
        o_ref[...] = jnp.maximum(
            a0_ref[...] + a1_ref[...] + h_ref[...] * d2 + b_ref[...], 0.0)

    return pl.pallas_call(
        body,
        grid=(_NRB,),
        in_specs=[
            pl.BlockSpec((_RB, HID), lambda i: (i, 0)),
            pl.BlockSpec((_RB, HID), lambda i: (i, 0)),
            pl.BlockSpec((_RB, HID), lambda i: (i, 0)),
            pl.BlockSpec((_RB, 1), lambda i: (i, 0)),
            pl.BlockSpec((1, HID), lambda i: (0, 0)),
        ],
        out_specs=pl.BlockSpec((_RB, HID), lambda i: (i, 0)),
        out_shape=jax.ShapeDtypeStruct((N, HID), jnp.float32),
    )(a0, a1, hlin, dinv, b1)


def _tc_out(a0, a1, h, dinv, W2, b2):
    def body(a0_ref, a1_ref, h_ref, d_ref, w_ref, b_ref, o_ref):
        d2 = d_ref[...] * d_ref[...]
        agg = a0_ref[...] + a1_ref[...] + h_ref[...] * d2
        o_ref[...] = jnp.dot(agg, w_ref[...],
                             preferred_element_type=jnp.float32) + b_ref[...]

    return pl.pallas_call(
        body,
        grid=(_NRB,),
        in_specs=[
            pl.BlockSpec((_RB, HID), lambda i: (i, 0)),
            pl.BlockSpec((_RB, HID), lambda i: (i, 0)),
            pl.BlockSpec((_RB, HID), lambda i: (i, 0)),
            pl.BlockSpec((_RB, 1), lambda i: (i, 0)),
            pl.BlockSpec((HID, D_OUT), lambda i: (0, 0)),
            pl.BlockSpec((1, D_OUT), lambda i: (0, 0)),
        ],
        out_specs=pl.BlockSpec((_RB, D_OUT), lambda i: (i, 0)),
        out_shape=jax.ShapeDtypeStruct((N, D_OUT), jnp.float32),
    )(a0, a1, h, dinv, W2, b2)


def kernel(x, edge_index, edge_weight, W1, b1, W2, b2):
    row = edge_index[0]
    col = edge_index[1]
    zeros1 = jnp.zeros((ROWS_PT,), jnp.float32)
    zeros16 = jnp.zeros((ROWS_PT, HID), jnp.float32)

    deg0, deg1 = _deg_kernel(col, edge_weight, zeros1)
    dinv = _tc_dinv(deg0, deg1)
    norm = _norm_kernel(row, col, edge_weight, dinv)

    hlin = _tc_mm1(x, W1)
    a0, a1 = _agg_kernel(row, col, norm, hlin, zeros16)

    dinv_n = dinv.reshape(NP_, 1)
    h = _tc_relu_combine(a0, a1, hlin, dinv_n, b1.reshape(1, HID))

    a20, a21 = _agg_kernel(row, col, norm, h, zeros16)
    out = _tc_out(a20, a21, h, dinv_n, W2, b2.reshape(1, D_OUT))
    return out


# speedup vs baseline: 1.3463x; 1.3463x over previous
import functools

import jax
import jax.numpy as jnp
from jax import lax
from jax.experimental import pallas as pl
from jax.experimental.pallas import tpu as pltpu
from jax.experimental.pallas import tpu_sc as plsc

N = 100000
E = 1600000
D_IN = 128
HID = 16
D_OUT = 40

NP_ = 102400
NC = 2
NS = 16
NW = NC * NS
ET = E // NW
EC = E // NC
CHUNK_D = 5000
CHUNK_N = 2000
CHUNK_A = 400
NCH_D = ET // CHUNK_D
NCH_N = ET // CHUNK_N
NCH_A = ET // CHUNK_A
ROWS_PT = NP_ // NS

_mesh = plsc.VectorSubcoreMesh(core_axis_name="c", subcore_axis_name="s")


def _f32(*shape):
    return jax.ShapeDtypeStruct(shape, jnp.float32)


@functools.partial(
    pl.kernel,
    out_type=(_f32(NP_), _f32(NP_)),
    mesh=_mesh,
    compiler_params=pltpu.CompilerParams(needs_layout_passes=False, use_tc_tiling_on_sc=False),
    scratch_types=[
        pltpu.VMEM((CHUNK_D,), jnp.int32),
        pltpu.VMEM((CHUNK_D,), jnp.float32),
        pltpu.VMEM_SHARED((NP_,), jnp.float32),
    ],
)
def _deg_kernel(col_hbm, ew_hbm, zeros_hbm, deg0_hbm, deg1_hbm,
                col_v, ew_v, acc):
    c = lax.axis_index("c")
    s = lax.axis_index("s")
    pltpu.sync_copy(zeros_hbm.at[pl.ds(0, ROWS_PT)],
                    acc.at[pl.ds(s * ROWS_PT, ROWS_PT)])
    plsc.subcore_barrier()

    base_t = c * EC + s * ET

    def body(i, carry):
        base = base_t + i * CHUNK_D
        pltpu.sync_copy(col_hbm.at[pl.ds(base, CHUNK_D)], col_v)
        pltpu.sync_copy(ew_hbm.at[pl.ds(base, CHUNK_D)], ew_v)
        pltpu.sync_copy(ew_v, acc.at[col_v], add=True)
        return carry

    lax.fori_loop(0, NCH_D, body, 0)
    plsc.subcore_barrier()

    @pl.when(c == 0)
    def _():
        pltpu.sync_copy(acc.at[pl.ds(s * ROWS_PT, ROWS_PT)],
                        deg0_hbm.at[pl.ds(s * ROWS_PT, ROWS_PT)])

    @pl.when(c == 1)
    def _():
        pltpu.sync_copy(acc.at[pl.ds(s * ROWS_PT, ROWS_PT)],
                        deg1_hbm.at[pl.ds(s * ROWS_PT, ROWS_PT)])


@functools.partial(
    pl.kernel,
    out_type=_f32(E),
    mesh=_mesh,
    compiler_params=pltpu.CompilerParams(needs_layout_passes=False, use_tc_tiling_on_sc=False),
    scratch_types=[
        pltpu.VMEM((NP_,), jnp.float32),
        pltpu.VMEM((CHUNK_N,), jnp.int32),
        pltpu.VMEM((CHUNK_N,), jnp.int32),
        pltpu.VMEM((CHUNK_N,), jnp.float32),
        pltpu.VMEM((CHUNK_N,), jnp.float32),
    ],
)
def _norm_kernel(row_hbm, col_hbm, ew_hbm, dinv_hbm, norm_hbm,
                 dinv_v, row_v, col_v, ew_v, norm_v):
    c = lax.axis_index("c")
    s = lax.axis_index("s")
    pltpu.sync_copy(dinv_hbm, dinv_v)
    base_t = c * EC + s * ET

    def body(i, carry):
        base = base_t + i * CHUNK_N
        pltpu.sync_copy(row_hbm.at[pl.ds(base, CHUNK_N)], row_v)
        pltpu.sync_copy(col_hbm.at[pl.ds(base, CHUNK_N)], col_v)
        pltpu.sync_copy(ew_hbm.at[pl.ds(base, CHUNK_N)], ew_v)

        def inner(g, carry2):
            off = g * 16
            r16 = row_v[pl.ds(off, 16)]
            c16 = col_v[pl.ds(off, 16)]
            e16 = ew_v[pl.ds(off, 16)]
            dr = plsc.load_gather(dinv_v, [r16])
            dc = plsc.load_gather(dinv_v, [c16])
            norm_v[pl.ds(off, 16)] = dr * e16 * dc
            return carry2

        lax.fori_loop(0, CHUNK_N // 16, inner, 0, unroll=4)
        pltpu.sync_copy(norm_v, norm_hbm.at[pl.ds(base, CHUNK_N)])
        return carry

    lax.fori_loop(0, NCH_N, body, 0)


@functools.partial(
    pl.kernel,
    out_type=(_f32(NP_, HID), _f32(NP_, HID)),
    mesh=_mesh,
    compiler_params=pltpu.CompilerParams(needs_layout_passes=False, use_tc_tiling_on_sc=False),
    scratch_types=[
        pltpu.VMEM((CHUNK_A,), jnp.int32),
        pltpu.VMEM((CHUNK_A,), jnp.int32),
        pltpu.VMEM((CHUNK_A,), jnp.float32),
        pltpu.VMEM((CHUNK_A, HID), jnp.float32),
        pltpu.VMEM((CHUNK_A,), jnp.int32),
        pltpu.VMEM((CHUNK_A,), jnp.int32),
        pltpu.VMEM((CHUNK_A,), jnp.float32),
        pltpu.VMEM((CHUNK_A, HID), jnp.float32),
        pltpu.VMEM_SHARED((NP_, HID), jnp.float32),
        pltpu.SemaphoreType.DMA,
        pltpu.SemaphoreType.DMA,
        pltpu.SemaphoreType.DMA,
        pltpu.SemaphoreType.DMA,
    ],
)
def _agg_kernel(row_hbm, col_hbm, norm_hbm, tab_hbm, zeros_hbm,
                out0_hbm, out1_hbm,
                row_v0, col_v0, norm_v0, rows_v0,
                row_v1, col_v1, norm_v1, rows_v1,
                acc, semg0, semg1, sems0, sems1):
    c = lax.axis_index("c")
    s = lax.axis_index("s")
    pltpu.sync_copy(zeros_hbm.at[pl.ds(0, ROWS_PT)],
                    acc.at[pl.ds(s * ROWS_PT, ROWS_PT)])
    plsc.subcore_barrier()

    base_t = c * EC + s * ET
    slots = ((row_v0, col_v0, norm_v0, rows_v0, semg0, sems0),
             (row_v1, col_v1, norm_v1, rows_v1, semg1, sems1))

    def load_and_gather(i, slot):
        row_v, col_v, norm_v, rows_v, semg, _ = slot
        base = base_t + i * CHUNK_A
        pltpu.sync_copy(row_hbm.at[pl.ds(base, CHUNK_A)], row_v)
        pltpu.sync_copy(col_hbm.at[pl.ds(base, CHUNK_A)], col_v)
        pltpu.sync_copy(norm_hbm.at[pl.ds(base, CHUNK_A)], norm_v)
        pltpu.async_copy(tab_hbm.at[row_v], rows_v, semg)

    load_and_gather(0, slots[0])
    load_and_gather(1, slots[1])

    def body(i, carry):
        def step(slot):
            row_v, col_v, norm_v, rows_v, semg, sems = slot
            pltpu.make_async_copy(tab_hbm.at[row_v], rows_v, semg).wait()

            def scale(g, carry2):
                off = g * 16
                n16 = norm_v[pl.ds(off, 16)]
                for k in range(16):
                    rows_v[off + k] = rows_v[off + k] * n16[k]
                return carry2

            lax.fori_loop(0, CHUNK_A // 16, scale, 0, unroll=2)
            pltpu.async_copy(rows_v, acc.at[col_v], sems, add=True)

            @pl.when(i + 2 < NCH_A)
            def _():
                pltpu.make_async_copy(rows_v, acc.at[col_v], sems).wait()
                load_and_gather(i + 2, slot)

        @pl.when(i % 2 == 0)
        def _():
            step(slots[0])

        @pl.when(i % 2 == 1)
        def _():
            step(slots[1])

        return carry

    lax.fori_loop(0, NCH_A, body, 0)
    pltpu.make_async_copy(rows_v0, acc.at[col_v0], sems0).wait()
    pltpu.make_async_copy(rows_v1, acc.at[col_v1], sems1).wait()
    plsc.subcore_barrier()

    @pl.when(c == 0)
    def _():
        pltpu.sync_copy(acc.at[pl.ds(s * ROWS_PT, ROWS_PT)],
                        out0_hbm.at[pl.ds(s * ROWS_PT, ROWS_PT)])

    @pl.when(c == 1)
    def _():
        pltpu.sync_copy(acc.at[pl.ds(s * ROWS_PT, ROWS_PT)],
                        out1_hbm.at[pl.ds(s * ROWS_PT, ROWS_PT)])


def _tc_dinv(deg0, deg1):
    def body(d0_ref, d1_ref, o_ref):
        deg = d0_ref[...] + d1_ref[...] + 1.0
        o_ref[...] = lax.rsqrt(deg)

    return pl.pallas_call(
        body,
        out_shape=jax.ShapeDtypeStruct((NP_ // 128, 128), jnp.float32),
    )(deg0.reshape(NP_ // 128, 128), deg1.reshape(NP_ // 128, 128)).reshape(NP_)


_RB = 5000
_NRB = N // _RB


def _tc_mm1(x, W1):
    def body(x_ref, w_ref, o_ref):
        o_ref[...] = jnp.dot(x_ref[...], w_ref[...],
                             preferred_element_type=jnp.float32)

    return pl.pallas_call(
        body,
        grid=(_NRB,),
        in_specs=[
            pl.BlockSpec((_RB, D_IN), lambda i: (i, 0)),
            pl.BlockSpec((D_IN, HID), lambda i: (0, 0)),
        ],
        out_specs=pl.BlockSpec((_RB, HID), lambda i: (i, 0)),
        out_shape=jax.ShapeDtypeStruct((N, HID), jnp.float32),
    )(x, W1)


def _tc_relu_combine(a0, a1, hlin, dinv, b1):
    def body(a0_ref, a1_ref, h_ref, d_ref, b_ref, o_ref):
        d2 = d_ref[...] * d_ref[...]
        o_ref[...] = jnp.maximum(
            a0_ref[...] + a1_ref[...] + h_ref[...] * d2 + b_ref[...], 0.0)

    return pl.pallas_call(
        body,
        grid=(_NRB,),
        in_specs=[
            pl.BlockSpec((_RB, HID), lambda i: (i, 0)),
            pl.BlockSpec((_RB, HID), lambda i: (i, 0)),
            pl.BlockSpec((_RB, HID), lambda i: (i, 0)),
            pl.BlockSpec((_RB, 1), lambda i: (i, 0)),
            pl.BlockSpec((1, HID), lambda i: (0, 0)),
        ],
        out_specs=pl.BlockSpec((_RB, HID), lambda i: (i, 0)),
        out_shape=jax.ShapeDtypeStruct((N, HID), jnp.float32),
    )(a0, a1, hlin, dinv, b1)


def _tc_out(a0, a1, h, dinv, W2, b2):
    def body(a0_ref, a1_ref, h_ref, d_ref, w_ref, b_ref, o_ref):
        d2 = d_ref[...] * d_ref[...]
        agg = a0_ref[...] + a1_ref[...] + h_ref[...] * d2
        o_ref[...] = jnp.dot(agg, w_ref[...],
                             preferred_element_type=jnp.float32) + b_ref[...]

    return pl.pallas_call(
        body,
        grid=(_NRB,),
        in_specs=[
            pl.BlockSpec((_RB, HID), lambda i: (i, 0)),
            pl.BlockSpec((_RB, HID), lambda i: (i, 0)),
            pl.BlockSpec((_RB, HID), lambda i: (i, 0)),
            pl.BlockSpec((_RB, 1), lambda i: (i, 0)),
            pl.BlockSpec((HID, D_OUT), lambda i: (0, 0)),
            pl.BlockSpec((1, D_OUT), lambda i: (0, 0)),
        ],
        out_specs=pl.BlockSpec((_RB, D_OUT), lambda i: (i, 0)),
        out_shape=jax.ShapeDtypeStruct((N, D_OUT), jnp.float32),
    )(a0, a1, h, dinv, W2, b2)


def kernel(x, edge_index, edge_weight, W1, b1, W2, b2):
    row = edge_index[0]
    col = edge_index[1]
    zeros1 = jnp.zeros((ROWS_PT,), jnp.float32)
    zeros16 = jnp.zeros((ROWS_PT, HID), jnp.float32)

    deg0, deg1 = _deg_kernel(col, edge_weight, zeros1)
    dinv = _tc_dinv(deg0, deg1)
    norm = _norm_kernel(row, col, edge_weight, dinv)

    hlin = _tc_mm1(x, W1)
    a0, a1 = _agg_kernel(row, col, norm, hlin, zeros16)

    dinv_n = dinv.reshape(NP_, 1)
    h = _tc_relu_combine(a0, a1, hlin, dinv_n, b1.reshape(1, HID))

    a20, a21 = _agg_kernel(row, col, norm, h, zeros16)
    out = _tc_out(a20, a21, h, dinv_n, W2, b2.reshape(1, D_OUT))
    return out


# speedup vs baseline: 1.3700x; 1.0176x over previous
import functools

import jax
import jax.numpy as jnp
from jax import lax
from jax.experimental import pallas as pl
from jax.experimental.pallas import tpu as pltpu
from jax.experimental.pallas import tpu_sc as plsc

N = 100000
E = 1600000
D_IN = 128
HID = 16
D_OUT = 40

NP_ = 102400
NC = 2
NS = 16
NW = NC * NS
ET = E // NW
EC = E // NC
CHUNK_D = 5000
CHUNK_N = 2000
CHUNK_A = 400
NCH_D = ET // CHUNK_D
NCH_N = ET // CHUNK_N
NCH_A = ET // CHUNK_A
ROWS_PT = NP_ // NS

_mesh = plsc.VectorSubcoreMesh(core_axis_name="c", subcore_axis_name="s")
_sc_params = pltpu.CompilerParams(needs_layout_passes=False,
                                  use_tc_tiling_on_sc=False)


def _f32(*shape):
    return jax.ShapeDtypeStruct(shape, jnp.float32)


@functools.partial(
    pl.kernel,
    out_type=(_f32(NP_), _f32(NP_)),
    mesh=_mesh,
    compiler_params=_sc_params,
    scratch_types=[
        pltpu.VMEM((CHUNK_D,), jnp.int32),
        pltpu.VMEM((CHUNK_D,), jnp.float32),
        pltpu.VMEM_SHARED((NP_,), jnp.float32),
    ],
)
def _deg_kernel(col_hbm, ew_hbm, zeros_hbm, deg0_hbm, deg1_hbm,
                col_v, ew_v, acc):
    c = lax.axis_index("c")
    s = lax.axis_index("s")
    pltpu.sync_copy(zeros_hbm.at[pl.ds(0, ROWS_PT)],
                    acc.at[pl.ds(s * ROWS_PT, ROWS_PT)])
    plsc.subcore_barrier()

    base_t = c * EC + s * ET

    def body(i, carry):
        base = base_t + i * CHUNK_D
        pltpu.sync_copy(col_hbm.at[pl.ds(base, CHUNK_D)], col_v)
        pltpu.sync_copy(ew_hbm.at[pl.ds(base, CHUNK_D)], ew_v)
        pltpu.sync_copy(ew_v, acc.at[col_v], add=True)
        return carry

    lax.fori_loop(0, NCH_D, body, 0)
    plsc.subcore_barrier()

    @pl.when(c == 0)
    def _():
        pltpu.sync_copy(acc.at[pl.ds(s * ROWS_PT, ROWS_PT)],
                        deg0_hbm.at[pl.ds(s * ROWS_PT, ROWS_PT)])

    @pl.when(c == 1)
    def _():
        pltpu.sync_copy(acc.at[pl.ds(s * ROWS_PT, ROWS_PT)],
                        deg1_hbm.at[pl.ds(s * ROWS_PT, ROWS_PT)])


@functools.partial(
    pl.kernel,
    out_type=_f32(E),
    mesh=_mesh,
    compiler_params=_sc_params,
    scratch_types=[
        pltpu.VMEM((NP_,), jnp.float32),
        pltpu.VMEM((CHUNK_N,), jnp.int32),
        pltpu.VMEM((CHUNK_N,), jnp.int32),
        pltpu.VMEM((CHUNK_N,), jnp.float32),
        pltpu.VMEM((CHUNK_N,), jnp.float32),
        pltpu.VMEM((CHUNK_N,), jnp.int32),
        pltpu.VMEM((CHUNK_N,), jnp.int32),
        pltpu.VMEM((CHUNK_N,), jnp.float32),
        pltpu.VMEM((CHUNK_N,), jnp.float32),
        pltpu.SemaphoreType.DMA,
        pltpu.SemaphoreType.DMA,
        pltpu.SemaphoreType.DMA,
        pltpu.SemaphoreType.DMA,
    ],
)
def _norm_kernel(row_hbm, col_hbm, ew_hbm, dinv_hbm, norm_hbm,
                 dinv_v,
                 row_v0, col_v0, ew_v0, norm_v0,
                 row_v1, col_v1, ew_v1, norm_v1,
                 semi0, semi1, semo0, semo1):
    c = lax.axis_index("c")
    s = lax.axis_index("s")
    pltpu.sync_copy(dinv_hbm, dinv_v)
    base_t = c * EC + s * ET
    slots = ((row_v0, col_v0, ew_v0, norm_v0, semi0, semo0),
             (row_v1, col_v1, ew_v1, norm_v1, semi1, semo1))

    def prefetch(i, slot):
        row_v, col_v, ew_v, _, semi, _ = slot
        base = base_t + i * CHUNK_N
        pltpu.async_copy(row_hbm.at[pl.ds(base, CHUNK_N)], row_v, semi)
        pltpu.async_copy(col_hbm.at[pl.ds(base, CHUNK_N)], col_v, semi)
        pltpu.async_copy(ew_hbm.at[pl.ds(base, CHUNK_N)], ew_v, semi)

    prefetch(0, slots[0])
    prefetch(1, slots[1])

    def body(i, carry):
        def step(slot):
            row_v, col_v, ew_v, norm_v, semi, semo = slot
            base = base_t + i * CHUNK_N
            pltpu.make_async_copy(row_hbm.at[pl.ds(base, CHUNK_N)], row_v, semi).wait()
            pltpu.make_async_copy(col_hbm.at[pl.ds(base, CHUNK_N)], col_v, semi).wait()
            pltpu.make_async_copy(ew_hbm.at[pl.ds(base, CHUNK_N)], ew_v, semi).wait()

            @pl.when(i >= 2)
            def _():
                prev = base_t + (i - 2) * CHUNK_N
                pltpu.make_async_copy(norm_v, norm_hbm.at[pl.ds(prev, CHUNK_N)], semo).wait()

            @plsc.parallel_loop(0, CHUNK_N // 16, unroll=4)
            def inner(g):
                off = g * 16
                r16 = row_v[pl.ds(off, 16)]
                c16 = col_v[pl.ds(off, 16)]
                e16 = ew_v[pl.ds(off, 16)]
                dr = plsc.load_gather(dinv_v, [r16])
                dc = plsc.load_gather(dinv_v, [c16])
                norm_v[pl.ds(off, 16)] = dr * e16 * dc

            pltpu.async_copy(norm_v, norm_hbm.at[pl.ds(base, CHUNK_N)], semo)

            @pl.when(i + 2 < NCH_N)
            def _():
                prefetch(i + 2, slot)

        @pl.when(i % 2 == 0)
        def _():
            step(slots[0])

        @pl.when(i % 2 == 1)
        def _():
            step(slots[1])

        return carry

    lax.fori_loop(0, NCH_N, body, 0)
    last0 = base_t + (NCH_N - 2) * CHUNK_N
    last1 = base_t + (NCH_N - 1) * CHUNK_N
    pltpu.make_async_copy(norm_v0, norm_hbm.at[pl.ds(last0, CHUNK_N)], semo0).wait()
    pltpu.make_async_copy(norm_v1, norm_hbm.at[pl.ds(last1, CHUNK_N)], semo1).wait()


@functools.partial(
    pl.kernel,
    out_type=(_f32(NP_, HID), _f32(NP_, HID)),
    mesh=_mesh,
    compiler_params=_sc_params,
    scratch_types=[
        pltpu.VMEM((CHUNK_A,), jnp.int32),
        pltpu.VMEM((CHUNK_A,), jnp.int32),
        pltpu.VMEM((CHUNK_A,), jnp.float32),
        pltpu.VMEM((CHUNK_A, HID), jnp.float32),
        pltpu.VMEM((CHUNK_A,), jnp.int32),
        pltpu.VMEM((CHUNK_A,), jnp.int32),
        pltpu.VMEM((CHUNK_A,), jnp.float32),
        pltpu.VMEM((CHUNK_A, HID), jnp.float32),
        pltpu.VMEM((CHUNK_A,), jnp.int32),
        pltpu.VMEM((CHUNK_A,), jnp.int32),
        pltpu.VMEM((CHUNK_A,), jnp.float32),
        pltpu.VMEM((CHUNK_A, HID), jnp.float32),
        pltpu.VMEM_SHARED((NP_, HID), jnp.float32),
        pltpu.SemaphoreType.DMA,
        pltpu.SemaphoreType.DMA,
        pltpu.SemaphoreType.DMA,
        pltpu.SemaphoreType.DMA,
        pltpu.SemaphoreType.DMA,
        pltpu.SemaphoreType.DMA,
    ],
)
def _agg_kernel(row_hbm, col_hbm, norm_hbm, tab_hbm, zeros_hbm,
                out0_hbm, out1_hbm,
                row_v0, col_v0, norm_v0, rows_v0,
                row_v1, col_v1, norm_v1, rows_v1,
                row_v2, col_v2, norm_v2, rows_v2,
                acc, semg0, semg1, semg2, sems0, sems1, sems2):
    c = lax.axis_index("c")
    s = lax.axis_index("s")
    pltpu.sync_copy(zeros_hbm.at[pl.ds(0, ROWS_PT)],
                    acc.at[pl.ds(s * ROWS_PT, ROWS_PT)])
    plsc.subcore_barrier()

    base_t = c * EC + s * ET
    slots = ((row_v0, col_v0, norm_v0, rows_v0, semg0, sems0),
             (row_v1, col_v1, norm_v1, rows_v1, semg1, sems1),
             (row_v2, col_v2, norm_v2, rows_v2, semg2, sems2))

    def load_and_gather(i, slot):
        row_v, col_v, norm_v, rows_v, semg, _ = slot
        base = base_t + i * CHUNK_A
        pltpu.sync_copy(row_hbm.at[pl.ds(base, CHUNK_A)], row_v)
        pltpu.sync_copy(col_hbm.at[pl.ds(base, CHUNK_A)], col_v)
        pltpu.sync_copy(norm_hbm.at[pl.ds(base, CHUNK_A)], norm_v)
        pltpu.async_copy(tab_hbm.at[row_v], rows_v, semg)

    load_and_gather(0, slots[0])
    load_and_gather(1, slots[1])

    def body(i, carry):
        def prep(slot):
            _, col_v, _, rows_v, _, sems = slot

            @pl.when(i >= 1)
            def _():
                pltpu.make_async_copy(rows_v, acc.at[col_v], sems).wait()

            load_and_gather(i + 2, slot)

        @pl.when(jnp.logical_and(i + 2 < NCH_A, (i + 2) % 3 == 0))
        def _():
            prep(slots[0])

        @pl.when(jnp.logical_and(i + 2 < NCH_A, (i + 2) % 3 == 1))
        def _():
            prep(slots[1])

        @pl.when(jnp.logical_and(i + 2 < NCH_A, (i + 2) % 3 == 2))
        def _():
            prep(slots[2])

        def step(slot):
            row_v, col_v, norm_v, rows_v, semg, sems = slot
            pltpu.make_async_copy(tab_hbm.at[row_v], rows_v, semg).wait()

            @plsc.parallel_loop(0, CHUNK_A // 16, unroll=4)
            def scale(g):
                off = g * 16
                n16 = norm_v[pl.ds(off, 16)]
                for k in range(16):
                    rows_v[off + k] = rows_v[off + k] * n16[k]

            pltpu.async_copy(rows_v, acc.at[col_v], sems, add=True)

        @pl.when(i % 3 == 0)
        def _():
            step(slots[0])

        @pl.when(i % 3 == 1)
        def _():
            step(slots[1])

        @pl.when(i % 3 == 2)
        def _():
            step(slots[2])

        return carry

    lax.fori_loop(0, NCH_A, body, 0)
    pltpu.make_async_copy(rows_v0, acc.at[col_v0], sems0).wait()
    pltpu.make_async_copy(rows_v1, acc.at[col_v1], sems1).wait()
    pltpu.make_async_copy(rows_v2, acc.at[col_v2], sems2).wait()
    plsc.subcore_barrier()

    @pl.when(c == 0)
    def _():
        pltpu.sync_copy(acc.at[pl.ds(s * ROWS_PT, ROWS_PT)],
                        out0_hbm.at[pl.ds(s * ROWS_PT, ROWS_PT)])

    @pl.when(c == 1)
    def _():
        pltpu.sync_copy(acc.at[pl.ds(s * ROWS_PT, ROWS_PT)],
                        out1_hbm.at[pl.ds(s * ROWS_PT, ROWS_PT)])


_RB = 5000
_NRB = N // _RB


def _tc_mm1_dinv(x, W1, deg0, deg1):
    def body(x_ref, w_ref, d0_ref, d1_ref, o_ref, di_ref):
        o_ref[...] = jnp.dot(x_ref[...], w_ref[...],
                             preferred_element_type=jnp.float32)
        deg = d0_ref[...] + d1_ref[...] + 1.0
        di_ref[...] = lax.rsqrt(deg)

    return pl.pallas_call(
        body,
        grid=(_NRB,),
        in_specs=[
            pl.BlockSpec((_RB, D_IN), lambda i: (i, 0)),
            pl.BlockSpec((D_IN, HID), lambda i: (0, 0)),
            pl.BlockSpec((NP_ // 128, 128), lambda i: (0, 0)),
            pl.BlockSpec((NP_ // 128, 128), lambda i: (0, 0)),
        ],
        out_specs=[
            pl.BlockSpec((_RB, HID), lambda i: (i, 0)),
            pl.BlockSpec((NP_ // 128, 128), lambda i: (0, 0)),
        ],
        out_shape=[
            jax.ShapeDtypeStruct((N, HID), jnp.float32),
            jax.ShapeDtypeStruct((NP_ // 128, 128), jnp.float32),
        ],
    )(x, W1, deg0.reshape(NP_ // 128, 128), deg1.reshape(NP_ // 128, 128))


def _tc_relu_combine(a0, a1, hlin, dinv, b1):
    def body(a0_ref, a1_ref, h_ref, d_ref, b_ref, o_ref):
        d2 = d_ref[...] * d_ref[...]
        o_ref[...] = jnp.maximum(
            a0_ref[...] + a1_ref[...] + h_ref[...] * d2 + b_ref[...], 0.0)

    return pl.pallas_call(
        body,
        grid=(_NRB,),
        in_specs=[
            pl.BlockSpec((_RB, HID), lambda i: (i, 0)),
            pl.BlockSpec((_RB, HID), lambda i: (i, 0)),
            pl.BlockSpec((_RB, HID), lambda i: (i, 0)),
            pl.BlockSpec((_RB, 1), lambda i: (i, 0)),
            pl.BlockSpec((1, HID), lambda i: (0, 0)),
        ],
        out_specs=pl.BlockSpec((_RB, HID), lambda i: (i, 0)),
        out_shape=jax.ShapeDtypeStruct((N, HID), jnp.float32),
    )(a0, a1, hlin, dinv, b1)


def _tc_out(a0, a1, h, dinv, W2, b2):
    def body(a0_ref, a1_ref, h_ref, d_ref, w_ref, b_ref, o_ref):
        d2 = d_ref[...] * d_ref[...]
        agg = a0_ref[...] + a1_ref[...] + h_ref[...] * d2
        o_ref[...] = jnp.dot(agg, w_ref[...],
                             preferred_element_type=jnp.float32) + b_ref[...]

    return pl.pallas_call(
        body,
        grid=(_NRB,),
        in_specs=[
            pl.BlockSpec((_RB, HID), lambda i: (i, 0)),
            pl.BlockSpec((_RB, HID), lambda i: (i, 0)),
            pl.BlockSpec((_RB, HID), lambda i: (i, 0)),
            pl.BlockSpec((_RB, 1), lambda i: (i, 0)),
            pl.BlockSpec((HID, D_OUT), lambda i: (0, 0)),
            pl.BlockSpec((1, D_OUT), lambda i: (0, 0)),
        ],
        out_specs=pl.BlockSpec((_RB, D_OUT), lambda i: (i, 0)),
        out_shape=jax.ShapeDtypeStruct((N, D_OUT), jnp.float32),
    )(a0, a1, h, dinv, W2, b2)


def kernel(x, edge_index, edge_weight, W1, b1, W2, b2):
    row = edge_index[0]
    col = edge_index[1]
    zeros1 = jnp.zeros((ROWS_PT,), jnp.float32)
    zeros16 = jnp.zeros((ROWS_PT, HID), jnp.float32)

    deg0, deg1 = _deg_kernel(col, edge_weight, zeros1)
    hlin, dinv2d = _tc_mm1_dinv(x, W1, deg0, deg1)
    dinv = dinv2d.reshape(NP_)
    norm = _norm_kernel(row, col, edge_weight, dinv)

    a0, a1 = _agg_kernel(row, col, norm, hlin, zeros16)

    dinv_n = dinv.reshape(NP_, 1)
    h = _tc_relu_combine(a0, a1, hlin, dinv_n, b1.reshape(1, HID))

    a20, a21 = _agg_kernel(row, col, norm, h, zeros16)
    out = _tc_out(a20, a21, h, dinv_n, W2, b2.reshape(1, D_OUT))
    return out
